# baseline (speedup 1.0000x reference)
import functools

import jax
import jax.numpy as jnp
from jax import lax
from jax.experimental import pallas as pl
from jax.experimental.pallas import tpu as pltpu
from jax.experimental.pallas import tpu_sc as plsc

N = 10000
E = 320000
D_IN = 128
D_OUT = 32
D2 = 2 * D_OUT

NC = 2
NS = 16
NW = NC * NS
B = 128
NB = 80
EPW = NB * B
E_PAD = NW * EPW
NA = N + 8

ROWS_MAIN = 632
ROWS_LAST = N - (NS - 1) * ROWS_MAIN

R = 2000
G = N // R

_mesh = plsc.VectorSubcoreMesh(core_axis_name="c", subcore_axis_name="s")
_sc_params = pltpu.CompilerParams(use_tc_tiling_on_sc=False)
_sc_params_nl = pltpu.CompilerParams(use_tc_tiling_on_sc=False,
                                     needs_layout_passes=False)


def _node_slice_copy(copy_fn, sid):
    @pl.when(sid != NS - 1)
    def _():
        copy_fn(sid * ROWS_MAIN, ROWS_MAIN)

    @pl.when(sid == NS - 1)
    def _():
        copy_fn((NS - 1) * ROWS_MAIN, ROWS_LAST)


@functools.partial(
    pl.kernel,
    out_type=jax.ShapeDtypeStruct((NC * N,), jnp.float32),
    mesh=_mesh,
    scratch_types=[
        pltpu.VMEM((NB, B), jnp.int32),
        pltpu.VMEM((B,), jnp.float32),
        pltpu.VMEM((ROWS_MAIN,), jnp.float32),
        pltpu.VMEM_SHARED((NA,), jnp.float32),
        pltpu.SemaphoreType.DMA,
    ],
    compiler_params=_sc_params,
)
def _deg_kernel(dst_hbm, zeros_hbm, out_hbm, dst_v, ones_v, stage_v,
                acc_sh, sem):
    cid = lax.axis_index("c")
    sid = lax.axis_index("s")
    wid = sid * NC + cid

    pltpu.sync_copy(zeros_hbm, stage_v)
    _node_slice_copy(
        lambda s, n: pltpu.sync_copy(stage_v.at[pl.ds(0, n)],
                                     acc_sh.at[pl.ds(s, n)]), sid)

    @pl.when(sid == 0)
    def _():
        pltpu.sync_copy(zeros_hbm.at[pl.ds(0, 8)], acc_sh.at[pl.ds(N, 8)])

    pltpu.sync_copy(dst_hbm.at[pl.ds(wid * NB, NB)], dst_v)
    for i in range(B // 16):
        ones_v[pl.ds(i * 16, 16)] = jnp.ones((16,), jnp.float32)
    plsc.subcore_barrier()

    K = 8

    def body(g, carry):
        for j in range(K):
            pltpu.async_copy(ones_v, acc_sh.at[dst_v.at[g * K + j]], sem,
                             add=True)
        for j in range(K):
            pltpu.make_async_copy(ones_v, acc_sh.at[dst_v.at[g * K + j]],
                                  sem).wait()
        return carry

    lax.fori_loop(0, NB // K, body, 0)
    plsc.subcore_barrier()

    def _writeout(s, n):
        pltpu.sync_copy(acc_sh.at[pl.ds(s, n)], stage_v.at[pl.ds(0, n)])
        pltpu.sync_copy(stage_v.at[pl.ds(0, n)],
                        out_hbm.at[pl.ds(cid * N + s, n)])

    _node_slice_copy(_writeout, sid)


CG = 4
NH = 2
HROWS = N // NH
ACC_LEN = HROWS * 16
EG = E // 4
BV = 128
NBT = 640
EGP = NBT * BV
CHB = 128
NCH = NBT // CHB
CHE = CHB * BV


def _bcast_lane(vec, j):
    idx = jnp.full((16, 1), j, jnp.int32)
    dnums = lax.GatherDimensionNumbers(
        offset_dims=(), collapsed_slice_dims=(0,), start_index_map=(0,))
    return lax.gather(vec, idx, dnums, slice_sizes=(1,),
                      mode=lax.GatherScatterMode.PROMISE_IN_BOUNDS)


@functools.partial(
    pl.kernel,
    out_type=jax.ShapeDtypeStruct((NW * ACC_LEN,), jnp.float32),
    mesh=_mesh,
    scratch_types=[
        pltpu.VMEM((ACC_LEN,), jnp.float32),
        pltpu.VMEM((CHE,), jnp.int32),
        pltpu.VMEM((CHE,), jnp.int32),
        pltpu.VMEM((BV,), jnp.int32),
        pltpu.VMEM((BV,), jnp.int32),
        pltpu.VMEM((BV, 16), jnp.float32),
        pltpu.VMEM((BV, 16), jnp.float32),
        pltpu.SemaphoreType.DMA,
        pltpu.SemaphoreType.DMA,
    ],
    compiler_params=_sc_params_nl,
)
def _prop_kernel(src_hbm, dst_hbm, feat4_hbm, out_hbm,
                 acc_v, srcc_v, dstc_v, idx0_v, idx1_v, gb0_v, gb1_v,
                 sem0, sem1):
    cid = lax.axis_index("c")
    sid = lax.axis_index("s")
    wid = sid * NC + cid
    g = sid % CG
    h = (sid // CG) % NH
    eg = sid // (CG * NH)
    q = cid * 2 + eg
    hoff = h * ACC_LEN
    iota16 = lax.iota(jnp.int32, 16)
    big = jnp.uint32(ACC_LEN)

    def zbody(i, carry):
        acc_v[pl.ds(pl.multiple_of(i * 16, 16), 16)] = jnp.zeros(
            (16,), jnp.float32)
        return carry

    lax.fori_loop(0, ACC_LEN // 16, zbody, 0)

    def comp_idx(idx_ref, b):
        for grp in range(8):
            o = pl.multiple_of(b * BV + grp * 16, 16)
            srcv = srcc_v[pl.ds(o, 16)]
            idx_ref[pl.ds(grp * 16, 16)] = srcv * 4 + g

    def process(gb_ref, b):
        for grp in range(8):
            o = pl.multiple_of(b * BV + grp * 16, 16)
            dstv = dstc_v[pl.ds(o, 16)]
            offv = dstv * 16 - hoff
            for j in range(16):
                base = _bcast_lane(offv, j)
                addr = base + iota16
                mask = plsc.bitcast(addr, jnp.uint32) < big
                vals = gb_ref[grp * 16 + j, :]
                plsc.addupdate_scatter(acc_v, [addr], vals, mask=mask)

    def chunk_body(c, carry):
        cb = q * EGP + c * CHE
        pltpu.sync_copy(src_hbm.at[pl.ds(cb, CHE)], srcc_v)
        pltpu.sync_copy(dst_hbm.at[pl.ds(cb, CHE)], dstc_v)
        comp_idx(idx0_v, 0)
        pltpu.async_copy(feat4_hbm.at[idx0_v], gb0_v, sem0)

        def pair_body(p, carry2):
            bA = 2 * p
            bB = 2 * p + 1
            comp_idx(idx1_v, bB)
            pltpu.async_copy(feat4_hbm.at[idx1_v], gb1_v, sem1)
            pltpu.make_async_copy(feat4_hbm.at[idx0_v], gb0_v, sem0).wait()
            process(gb0_v, bA)

            @pl.when(p < CHB // 2 - 1)
            def _():
                comp_idx(idx0_v, bA + 2)
                pltpu.async_copy(feat4_hbm.at[idx0_v], gb0_v, sem0)

            pltpu.make_async_copy(feat4_hbm.at[idx1_v], gb1_v, sem1).wait()
            process(gb1_v, bB)
            return carry2

        lax.fori_loop(0, CHB // 2, pair_body, 0)
        return carry

    lax.fori_loop(0, NCH, chunk_body, 0)
    pltpu.sync_copy(acc_v, out_hbm.at[pl.ds(wid * ACC_LEN, ACC_LEN)])


def _feat_body(x_ref, w_ref, b_ref, o_ref):
    y = jnp.dot(x_ref[...], w_ref[...],
                preferred_element_type=jnp.float32) + b_ref[...]
    xa = y[:, :D_OUT]
    h = y[:, D_OUT:]
    nrm = jnp.sqrt(jnp.sum(h * h, axis=1, keepdims=True))
    h = h * (1.8 / jnp.maximum(nrm, 1e-12))
    o_ref[...] = jnp.concatenate([xa, h], axis=1)


_feat_call = pl.pallas_call(
    _feat_body,
    grid=(G,),
    in_specs=[
        pl.BlockSpec((R, D_IN), lambda i: (i, 0)),
        pl.BlockSpec((D_IN, D2), lambda i: (0, 0)),
        pl.BlockSpec((1, D2), lambda i: (0, 0)),
    ],
    out_specs=pl.BlockSpec((R, D2), lambda i: (i, 0)),
    out_shape=jax.ShapeDtypeStruct((N, D2), jnp.float32),
)


def _scale_body(degT_ref, feat_ref, dsq_ref, fs_ref):
    d = degT_ref[:, 0:1] + degT_ref[:, 1:2] + 1.0
    dsq = lax.rsqrt(d)
    dsq_ref[...] = dsq
    fs_ref[...] = feat_ref[...] * dsq


_scale_call = pl.pallas_call(
    _scale_body,
    grid=(G,),
    in_specs=[
        pl.BlockSpec((R, 2), lambda i: (i, 0)),
        pl.BlockSpec((R, D2), lambda i: (i, 0)),
    ],
    out_specs=[
        pl.BlockSpec((R, 1), lambda i: (i, 0)),
        pl.BlockSpec((R, D2), lambda i: (i, 0)),
    ],
    out_shape=[
        jax.ShapeDtypeStruct((N, 1), jnp.float32),
        jax.ShapeDtypeStruct((N, D2), jnp.float32),
    ],
)


RC = 1000
GC = N // RC


def _combine_body(*refs):
    a = refs[:16]
    fs_ref, dsq_ref, ox_ref, oh_ref = refs[16:]
    cols = []
    for g in range(CG):
        t = (a[4 * g][0, 0] + a[4 * g + 1][0, 0]
             + a[4 * g + 2][0, 0] + a[4 * g + 3][0, 0])
        cols.append(t)
    t = jnp.concatenate(cols, axis=1)
    t = (t + fs_ref[...]) * dsq_ref[...]
    ox_ref[...] = t[:, :D_OUT]
    oh_ref[...] = t[:, D_OUT:]


def _acc_spec(g, e, c):
    return pl.BlockSpec(
        (1, 1, RC, 16),
        lambda i, g=g, e=e, c=c: (g + 4 * (i // 5) + 8 * e, c, i % 5, 0))


_combine_call = pl.pallas_call(
    _combine_body,
    grid=(GC,),
    in_specs=[_acc_spec(g, e, c)
              for g in range(CG) for e in range(2) for c in range(NC)]
    + [
        pl.BlockSpec((RC, D2), lambda i: (i, 0)),
        pl.BlockSpec((RC, 1), lambda i: (i, 0)),
    ],
    out_specs=[
        pl.BlockSpec((RC, D_OUT), lambda i: (i, 0)),
        pl.BlockSpec((RC, D_OUT), lambda i: (i, 0)),
    ],
    out_shape=[
        jax.ShapeDtypeStruct((N, D_OUT), jnp.float32),
        jax.ShapeDtypeStruct((N, D_OUT), jnp.float32),
    ],
)


def kernel(x, edge_index, W1, b1, W2, b2):
    src = edge_index[0]
    dst = edge_index[1]
    npad = E_PAD - E
    dst_p = jnp.concatenate(
        [dst, jnp.full((npad,), N, jnp.int32)]).reshape(NW * NB, B)
    src_q = jnp.pad(src.reshape(4, EG),
                    ((0, 0), (0, EGP - EG))).reshape(-1)
    dst_q = jnp.pad(dst.reshape(4, EG), ((0, 0), (0, EGP - EG)),
                    constant_values=2 ** 20).reshape(-1)
    W = jnp.concatenate([W1, W2], axis=1)
    b = jnp.concatenate([b1, b2])[None, :]

    feat = _feat_call(x, W, b)
    deg = _deg_kernel(dst_p, jnp.zeros((ROWS_MAIN,), jnp.float32))
    deg = deg.reshape(NC, N)
    dsq, fs = _scale_call(deg.T, feat)
    acc = _prop_kernel(src_q, dst_q, fs.reshape(CG * N, 16))
    acc = acc.reshape(NS, NC, HROWS, 16)
    args = [acc] * 16 + [fs, dsq]
    ox, oh = _combine_call(*args)
    return (oh, ox)


# speedup vs baseline: 1.1194x; 1.1194x over previous
import functools

import jax
import jax.numpy as jnp
from jax import lax
from jax.experimental import pallas as pl
from jax.experimental.pallas import tpu as pltpu
from jax.experimental.pallas import tpu_sc as plsc

N = 10000
E = 320000
D_IN = 128
D_OUT = 32
D2 = 2 * D_OUT

NC = 2
NS = 16
NW = NC * NS
B = 128
NB = 80
EPW = NB * B
E_PAD = NW * EPW
NA = N + 8

ROWS_MAIN = 632
ROWS_LAST = N - (NS - 1) * ROWS_MAIN

R = 2000
G = N // R

_mesh = plsc.VectorSubcoreMesh(core_axis_name="c", subcore_axis_name="s")
_sc_params = pltpu.CompilerParams(use_tc_tiling_on_sc=False)
_sc_params_nl = pltpu.CompilerParams(use_tc_tiling_on_sc=False,
                                     needs_layout_passes=False)


def _node_slice_copy(copy_fn, sid):
    @pl.when(sid != NS - 1)
    def _():
        copy_fn(sid * ROWS_MAIN, ROWS_MAIN)

    @pl.when(sid == NS - 1)
    def _():
        copy_fn((NS - 1) * ROWS_MAIN, ROWS_LAST)


@functools.partial(
    pl.kernel,
    out_type=jax.ShapeDtypeStruct((NC * N,), jnp.float32),
    mesh=_mesh,
    scratch_types=[
        pltpu.VMEM((NB, B), jnp.int32),
        pltpu.VMEM((B,), jnp.float32),
        pltpu.VMEM((ROWS_MAIN,), jnp.float32),
        pltpu.VMEM_SHARED((NA,), jnp.float32),
        pltpu.SemaphoreType.DMA,
    ],
    compiler_params=_sc_params,
)
def _deg_kernel(dst_hbm, zeros_hbm, out_hbm, dst_v, ones_v, stage_v,
                acc_sh, sem):
    cid = lax.axis_index("c")
    sid = lax.axis_index("s")
    wid = sid * NC + cid

    pltpu.sync_copy(zeros_hbm, stage_v)
    _node_slice_copy(
        lambda s, n: pltpu.sync_copy(stage_v.at[pl.ds(0, n)],
                                     acc_sh.at[pl.ds(s, n)]), sid)

    @pl.when(sid == 0)
    def _():
        pltpu.sync_copy(zeros_hbm.at[pl.ds(0, 8)], acc_sh.at[pl.ds(N, 8)])

    pltpu.sync_copy(dst_hbm.at[pl.ds(wid * NB, NB)], dst_v)
    for i in range(B // 16):
        ones_v[pl.ds(i * 16, 16)] = jnp.ones((16,), jnp.float32)
    plsc.subcore_barrier()

    K = 8

    def body(g, carry):
        for j in range(K):
            pltpu.async_copy(ones_v, acc_sh.at[dst_v.at[g * K + j]], sem,
                             add=True)
        for j in range(K):
            pltpu.make_async_copy(ones_v, acc_sh.at[dst_v.at[g * K + j]],
                                  sem).wait()
        return carry

    lax.fori_loop(0, NB // K, body, 0)
    plsc.subcore_barrier()

    def _writeout(s, n):
        pltpu.sync_copy(acc_sh.at[pl.ds(s, n)], stage_v.at[pl.ds(0, n)])
        pltpu.sync_copy(stage_v.at[pl.ds(0, n)],
                        out_hbm.at[pl.ds(cid * N + s, n)])

    _node_slice_copy(_writeout, sid)


CG = 4
NH = 2
HROWS = N // NH
ACC_LEN = HROWS * 16
EG = E // 4
BV = 128
NBT = 640
EGP = NBT * BV
CHB = 128
NCH = NBT // CHB
CHE = CHB * BV


def _bcast_lane(vec, j):
    idx = jnp.full((16, 1), j, jnp.int32)
    dnums = lax.GatherDimensionNumbers(
        offset_dims=(), collapsed_slice_dims=(0,), start_index_map=(0,))
    return lax.gather(vec, idx, dnums, slice_sizes=(1,),
                      mode=lax.GatherScatterMode.PROMISE_IN_BOUNDS)


@functools.partial(
    pl.kernel,
    out_type=jax.ShapeDtypeStruct((NW * ACC_LEN,), jnp.float32),
    mesh=_mesh,
    scratch_types=[
        pltpu.VMEM((ACC_LEN + 16,), jnp.float32),
        pltpu.VMEM((CHE,), jnp.int32),
        pltpu.VMEM((CHE,), jnp.int32),
        pltpu.VMEM((BV,), jnp.int32),
        pltpu.VMEM((BV,), jnp.int32),
        pltpu.VMEM((BV, 16), jnp.float32),
        pltpu.VMEM((BV, 16), jnp.float32),
        pltpu.SemaphoreType.DMA,
        pltpu.SemaphoreType.DMA,
    ],
    compiler_params=_sc_params_nl,
)
def _prop_kernel(src_hbm, dst_hbm, feat4_hbm, out_hbm,
                 acc_v, srcc_v, dstc_v, idx0_v, idx1_v, gb0_v, gb1_v,
                 sem0, sem1):
    cid = lax.axis_index("c")
    sid = lax.axis_index("s")
    wid = sid * NC + cid
    g = sid % CG
    h = (sid // CG) % NH
    eg = sid // (CG * NH)
    q = cid * 2 + eg
    hoff = h * ACC_LEN
    iota16 = lax.iota(jnp.int32, 16)
    big = jnp.uint32(ACC_LEN)

    def zbody(i, carry):
        acc_v[pl.ds(pl.multiple_of(i * 16, 16), 16)] = jnp.zeros(
            (16,), jnp.float32)
        return carry

    lax.fori_loop(0, ACC_LEN // 16 + 1, zbody, 0)

    def comp_idx(idx_ref, b):
        for grp in range(8):
            o = pl.multiple_of(b * BV + grp * 16, 16)
            srcv = srcc_v[pl.ds(o, 16)]
            idx_ref[pl.ds(grp * 16, 16)] = srcv * 4 + g

    def process(gb_ref, b):
        @plsc.parallel_loop(0, 8)
        def _(grp):
            o = pl.multiple_of(b * BV, 16) + grp * 16
            dstv = plsc.load_gather(dstc_v, [o + iota16])
            offv = dstv * 16 - hoff
            for j in range(16):
                base = _bcast_lane(offv, j)
                addr = base + iota16
                ok = plsc.bitcast(base, jnp.uint32) < big
                addr = jnp.where(ok, addr, ACC_LEN + iota16)
                vals = plsc.load_gather(
                    gb_ref, [jnp.full((16,), grp * 16 + j, jnp.int32),
                             iota16])
                plsc.addupdate_scatter(acc_v, [addr], vals)

    def chunk_body(c, carry):
        cb = q * EGP + c * CHE
        pltpu.sync_copy(src_hbm.at[pl.ds(cb, CHE)], srcc_v)
        pltpu.sync_copy(dst_hbm.at[pl.ds(cb, CHE)], dstc_v)
        comp_idx(idx0_v, 0)
        pltpu.async_copy(feat4_hbm.at[idx0_v], gb0_v, sem0)

        def pair_body(p, carry2):
            bA = 2 * p
            bB = 2 * p + 1
            comp_idx(idx1_v, bB)
            pltpu.async_copy(feat4_hbm.at[idx1_v], gb1_v, sem1)
            pltpu.make_async_copy(feat4_hbm.at[idx0_v], gb0_v, sem0).wait()
            process(gb0_v, bA)

            @pl.when(p < CHB // 2 - 1)
            def _():
                comp_idx(idx0_v, bA + 2)
                pltpu.async_copy(feat4_hbm.at[idx0_v], gb0_v, sem0)

            pltpu.make_async_copy(feat4_hbm.at[idx1_v], gb1_v, sem1).wait()
            process(gb1_v, bB)
            return carry2

        lax.fori_loop(0, CHB // 2, pair_body, 0)
        return carry

    lax.fori_loop(0, NCH, chunk_body, 0)
    pltpu.sync_copy(acc_v.at[pl.ds(0, ACC_LEN)],
                    out_hbm.at[pl.ds(wid * ACC_LEN, ACC_LEN)])


def _feat_body(x_ref, w_ref, b_ref, o_ref):
    y = jnp.dot(x_ref[...], w_ref[...],
                preferred_element_type=jnp.float32) + b_ref[...]
    xa = y[:, :D_OUT]
    h = y[:, D_OUT:]
    nrm = jnp.sqrt(jnp.sum(h * h, axis=1, keepdims=True))
    h = h * (1.8 / jnp.maximum(nrm, 1e-12))
    o_ref[...] = jnp.concatenate([xa, h], axis=1)


_feat_call = pl.pallas_call(
    _feat_body,
    grid=(G,),
    in_specs=[
        pl.BlockSpec((R, D_IN), lambda i: (i, 0)),
        pl.BlockSpec((D_IN, D2), lambda i: (0, 0)),
        pl.BlockSpec((1, D2), lambda i: (0, 0)),
    ],
    out_specs=pl.BlockSpec((R, D2), lambda i: (i, 0)),
    out_shape=jax.ShapeDtypeStruct((N, D2), jnp.float32),
)


def _scale_body(degT_ref, feat_ref, dsq_ref, fs_ref):
    d = degT_ref[:, 0:1] + degT_ref[:, 1:2] + 1.0
    dsq = lax.rsqrt(d)
    dsq_ref[...] = dsq
    fs_ref[...] = feat_ref[...] * dsq


_scale_call = pl.pallas_call(
    _scale_body,
    grid=(G,),
    in_specs=[
        pl.BlockSpec((R, 2), lambda i: (i, 0)),
        pl.BlockSpec((R, D2), lambda i: (i, 0)),
    ],
    out_specs=[
        pl.BlockSpec((R, 1), lambda i: (i, 0)),
        pl.BlockSpec((R, D2), lambda i: (i, 0)),
    ],
    out_shape=[
        jax.ShapeDtypeStruct((N, 1), jnp.float32),
        jax.ShapeDtypeStruct((N, D2), jnp.float32),
    ],
)


RC = 1000
GC = N // RC


def _combine_body(*refs):
    a = refs[:16]
    fs_ref, dsq_ref, ox_ref, oh_ref = refs[16:]
    cols = []
    for g in range(CG):
        t = (a[4 * g][0, 0] + a[4 * g + 1][0, 0]
             + a[4 * g + 2][0, 0] + a[4 * g + 3][0, 0])
        cols.append(t)
    t = jnp.concatenate(cols, axis=1)
    t = (t + fs_ref[...]) * dsq_ref[...]
    ox_ref[...] = t[:, :D_OUT]
    oh_ref[...] = t[:, D_OUT:]


def _acc_spec(g, e, c):
    return pl.BlockSpec(
        (1, 1, RC, 16),
        lambda i, g=g, e=e, c=c: (g + 4 * (i // 5) + 8 * e, c, i % 5, 0))


_combine_call = pl.pallas_call(
    _combine_body,
    grid=(GC,),
    in_specs=[_acc_spec(g, e, c)
              for g in range(CG) for e in range(2) for c in range(NC)]
    + [
        pl.BlockSpec((RC, D2), lambda i: (i, 0)),
        pl.BlockSpec((RC, 1), lambda i: (i, 0)),
    ],
    out_specs=[
        pl.BlockSpec((RC, D_OUT), lambda i: (i, 0)),
        pl.BlockSpec((RC, D_OUT), lambda i: (i, 0)),
    ],
    out_shape=[
        jax.ShapeDtypeStruct((N, D_OUT), jnp.float32),
        jax.ShapeDtypeStruct((N, D_OUT), jnp.float32),
    ],
)


def kernel(x, edge_index, W1, b1, W2, b2):
    src = edge_index[0]
    dst = edge_index[1]
    npad = E_PAD - E
    dst_p = jnp.concatenate(
        [dst, jnp.full((npad,), N, jnp.int32)]).reshape(NW * NB, B)
    src_q = jnp.pad(src.reshape(4, EG),
                    ((0, 0), (0, EGP - EG))).reshape(-1)
    dst_q = jnp.pad(dst.reshape(4, EG), ((0, 0), (0, EGP - EG)),
                    constant_values=2 ** 20).reshape(-1)
    W = jnp.concatenate([W1, W2], axis=1)
    b = jnp.concatenate([b1, b2])[None, :]

    feat = _feat_call(x, W, b)
    deg = _deg_kernel(dst_p, jnp.zeros((ROWS_MAIN,), jnp.float32))
    deg = deg.reshape(NC, N)
    dsq, fs = _scale_call(deg.T, feat)
    acc = _prop_kernel(src_q, dst_q, fs.reshape(CG * N, 16))
    acc = acc.reshape(NS, NC, HROWS, 16)
    args = [acc] * 16 + [fs, dsq]
    ox, oh = _combine_call(*args)
    return (oh, ox)


# speedup vs baseline: 2.2344x; 1.9960x over previous
import functools

import jax
import jax.numpy as jnp
from jax import lax
from jax.experimental import pallas as pl
from jax.experimental.pallas import tpu as pltpu
from jax.experimental.pallas import tpu_sc as plsc

N = 10000
E = 320000
D_IN = 128
D_OUT = 32
D2 = 2 * D_OUT

NC = 2
NS = 16
NW = NC * NS
B = 128
NB = 80
EPW = NB * B
E_PAD = NW * EPW
NA = N + 8

ROWS_MAIN = 632
ROWS_LAST = N - (NS - 1) * ROWS_MAIN
STG = 320
CHUNKS_MAIN = (320, 312)
CHUNKS_LAST = (320, 200)

R = 2000
G = N // R

_mesh = plsc.VectorSubcoreMesh(core_axis_name="c", subcore_axis_name="s")
_sc_params = pltpu.CompilerParams(use_tc_tiling_on_sc=False)


def _node_slice_copy(copy_fn, sid):
    @pl.when(sid != NS - 1)
    def _():
        copy_fn(sid * ROWS_MAIN, ROWS_MAIN)

    @pl.when(sid == NS - 1)
    def _():
        copy_fn((NS - 1) * ROWS_MAIN, ROWS_LAST)


@functools.partial(
    pl.kernel,
    out_type=jax.ShapeDtypeStruct((NC * N,), jnp.float32),
    mesh=_mesh,
    scratch_types=[
        pltpu.VMEM((NB, B), jnp.int32),
        pltpu.VMEM((B,), jnp.float32),
        pltpu.VMEM((ROWS_MAIN,), jnp.float32),
        pltpu.VMEM_SHARED((NA,), jnp.float32),
        pltpu.SemaphoreType.DMA,
    ],
    compiler_params=_sc_params,
)
def _deg_kernel(dst_hbm, zeros_hbm, out_hbm, dst_v, ones_v, stage_v,
                acc_sh, sem):
    cid = lax.axis_index("c")
    sid = lax.axis_index("s")
    wid = sid * NC + cid

    pltpu.sync_copy(zeros_hbm, stage_v)
    _node_slice_copy(
        lambda s, n: pltpu.sync_copy(stage_v.at[pl.ds(0, n)],
                                     acc_sh.at[pl.ds(s, n)]), sid)

    @pl.when(sid == 0)
    def _():
        pltpu.sync_copy(zeros_hbm.at[pl.ds(0, 8)], acc_sh.at[pl.ds(N, 8)])

    pltpu.sync_copy(dst_hbm.at[pl.ds(wid * NB, NB)], dst_v)
    for i in range(B // 16):
        ones_v[pl.ds(i * 16, 16)] = jnp.ones((16,), jnp.float32)
    plsc.subcore_barrier()

    K = 8

    def body(g, carry):
        for j in range(K):
            pltpu.async_copy(ones_v, acc_sh.at[dst_v.at[g * K + j]], sem,
                             add=True)
        for j in range(K):
            pltpu.make_async_copy(ones_v, acc_sh.at[dst_v.at[g * K + j]],
                                  sem).wait()
        return carry

    lax.fori_loop(0, NB // K, body, 0)
    plsc.subcore_barrier()

    def _writeout(s, n):
        pltpu.sync_copy(acc_sh.at[pl.ds(s, n)], stage_v.at[pl.ds(0, n)])
        pltpu.sync_copy(stage_v.at[pl.ds(0, n)],
                        out_hbm.at[pl.ds(cid * N + s, n)])

    _node_slice_copy(_writeout, sid)


@functools.partial(
    pl.kernel,
    out_type=jax.ShapeDtypeStruct((NC * N, D2), jnp.float32),
    mesh=_mesh,
    scratch_types=[
        pltpu.VMEM((NB, B), jnp.int32),
        pltpu.VMEM((NB, B), jnp.int32),
        pltpu.VMEM((B, D2), jnp.float32),
        pltpu.VMEM((B, D2), jnp.float32),
        pltpu.VMEM((B, D2), jnp.float32),
        pltpu.VMEM((B, D2), jnp.float32),
        pltpu.VMEM((STG, D2), jnp.float32),
        pltpu.VMEM_SHARED((NA, D2), jnp.float32),
        pltpu.SemaphoreType.DMA,
        pltpu.SemaphoreType.DMA,
        pltpu.SemaphoreType.DMA,
        pltpu.SemaphoreType.DMA,
        pltpu.SemaphoreType.DMA,
        pltpu.SemaphoreType.DMA,
        pltpu.SemaphoreType.DMA,
        pltpu.SemaphoreType.DMA,
    ],
    compiler_params=_sc_params,
)
def _prop_kernel(src_hbm, dst_hbm, feat_hbm, zeros_hbm, out_hbm,
                 src_v, dst_v, rows0_v, rows1_v, rows2_v, rows3_v,
                 stage_v, acc_sh,
                 gsem0, gsem1, gsem2, gsem3, ssem0, ssem1, ssem2, ssem3):
    cid = lax.axis_index("c")
    sid = lax.axis_index("s")
    wid = sid * NC + cid

    def _sub_chunks(copy_fn, sid):
        @pl.when(sid != NS - 1)
        def _():
            off = 0
            for n in CHUNKS_MAIN:
                copy_fn(sid * ROWS_MAIN + off, n)
                off += n

        @pl.when(sid == NS - 1)
        def _():
            off = 0
            for n in CHUNKS_LAST:
                copy_fn((NS - 1) * ROWS_MAIN + off, n)
                off += n

    pltpu.sync_copy(zeros_hbm, stage_v)
    _sub_chunks(
        lambda s, n: pltpu.sync_copy(stage_v.at[pl.ds(0, n)],
                                     acc_sh.at[pl.ds(s, n)]), sid)

    @pl.when(sid == 0)
    def _():
        pltpu.sync_copy(zeros_hbm.at[pl.ds(0, 8)], acc_sh.at[pl.ds(N, 8)])

    pltpu.sync_copy(src_hbm.at[pl.ds(wid * NB, NB)], src_v)
    pltpu.sync_copy(dst_hbm.at[pl.ds(wid * NB, NB)], dst_v)
    plsc.subcore_barrier()

    bufs = (rows0_v, rows1_v, rows2_v, rows3_v)
    gsems = (gsem0, gsem1, gsem2, gsem3)
    ssems = (ssem0, ssem1, ssem2, ssem3)
    for p in range(3):
        pltpu.async_copy(feat_hbm.at[src_v.at[p]], bufs[p], gsems[p])

    def body(k, carry):
        for p in range(4):
            i = 4 * k + p
            pltpu.make_async_copy(feat_hbm.at[src_v.at[i]], bufs[p],
                                  gsems[p]).wait()
            pltpu.async_copy(bufs[p], acc_sh.at[dst_v.at[i]], ssems[p],
                             add=True)
            pprev = (p + 3) % 4

            @pl.when(i + 3 < NB)
            def _(i=i, pprev=pprev):
                @pl.when(i >= 1)
                def _():
                    pltpu.make_async_copy(
                        bufs[pprev], acc_sh.at[dst_v.at[i - 1]],
                        ssems[pprev]).wait()
                pltpu.async_copy(feat_hbm.at[src_v.at[i + 3]], bufs[pprev],
                                 gsems[pprev])
        return carry

    lax.fori_loop(0, NB // 4, body, 0)
    for p in range(4):
        pltpu.make_async_copy(bufs[p], acc_sh.at[dst_v.at[NB - 4 + p]],
                              ssems[p]).wait()
    plsc.subcore_barrier()

    def _writeout(s, n):
        pltpu.sync_copy(acc_sh.at[pl.ds(s, n)], stage_v.at[pl.ds(0, n)])
        pltpu.sync_copy(stage_v.at[pl.ds(0, n)],
                        out_hbm.at[pl.ds(cid * N + s, n)])

    _sub_chunks(_writeout, sid)


def _feat_body(x_ref, w_ref, b_ref, o_ref):
    y = jnp.dot(x_ref[...], w_ref[...],
                preferred_element_type=jnp.float32) + b_ref[...]
    xa = y[:, :D_OUT]
    h = y[:, D_OUT:]
    nrm = jnp.sqrt(jnp.sum(h * h, axis=1, keepdims=True))
    h = h * (1.8 / jnp.maximum(nrm, 1e-12))
    o_ref[...] = jnp.concatenate([xa, h], axis=1)


_feat_call = pl.pallas_call(
    _feat_body,
    grid=(G,),
    in_specs=[
        pl.BlockSpec((R, D_IN), lambda i: (i, 0)),
        pl.BlockSpec((D_IN, D2), lambda i: (0, 0)),
        pl.BlockSpec((1, D2), lambda i: (0, 0)),
    ],
    out_specs=pl.BlockSpec((R, D2), lambda i: (i, 0)),
    out_shape=jax.ShapeDtypeStruct((N, D2), jnp.float32),
)


def _scale_body(degT_ref, feat_ref, dsq_ref, fs_ref):
    d = degT_ref[:, 0:1] + degT_ref[:, 1:2] + 1.0
    dsq = lax.rsqrt(d)
    dsq_ref[...] = dsq
    fs_ref[...] = feat_ref[...] * dsq


_scale_call = pl.pallas_call(
    _scale_body,
    grid=(G,),
    in_specs=[
        pl.BlockSpec((R, 2), lambda i: (i, 0)),
        pl.BlockSpec((R, D2), lambda i: (i, 0)),
    ],
    out_specs=[
        pl.BlockSpec((R, 1), lambda i: (i, 0)),
        pl.BlockSpec((R, D2), lambda i: (i, 0)),
    ],
    out_shape=[
        jax.ShapeDtypeStruct((N, 1), jnp.float32),
        jax.ShapeDtypeStruct((N, D2), jnp.float32),
    ],
)


def _combine_body(a0_ref, a1_ref, fs_ref, dsq_ref, ox_ref, oh_ref):
    t = (a0_ref[...] + a1_ref[...] + fs_ref[...]) * dsq_ref[...]
    ox_ref[...] = t[:, :D_OUT]
    oh_ref[...] = t[:, D_OUT:]


_combine_call = pl.pallas_call(
    _combine_body,
    grid=(G,),
    in_specs=[
        pl.BlockSpec((R, D2), lambda i: (i, 0)),
        pl.BlockSpec((R, D2), lambda i: (i, 0)),
        pl.BlockSpec((R, D2), lambda i: (i, 0)),
        pl.BlockSpec((R, 1), lambda i: (i, 0)),
    ],
    out_specs=[
        pl.BlockSpec((R, D_OUT), lambda i: (i, 0)),
        pl.BlockSpec((R, D_OUT), lambda i: (i, 0)),
    ],
    out_shape=[
        jax.ShapeDtypeStruct((N, D_OUT), jnp.float32),
        jax.ShapeDtypeStruct((N, D_OUT), jnp.float32),
    ],
)


def kernel(x, edge_index, W1, b1, W2, b2):
    src = edge_index[0]
    dst = edge_index[1]
    npad = E_PAD - E
    src_p = jnp.concatenate(
        [src, jnp.zeros((npad,), jnp.int32)]).reshape(NW * NB, B)
    dst_p = jnp.concatenate(
        [dst, jnp.full((npad,), N, jnp.int32)]).reshape(NW * NB, B)
    W = jnp.concatenate([W1, W2], axis=1)
    b = jnp.concatenate([b1, b2])[None, :]

    feat = _feat_call(x, W, b)
    deg = _deg_kernel(dst_p, jnp.zeros((ROWS_MAIN,), jnp.float32))
    deg = deg.reshape(NC, N)
    dsq, fs = _scale_call(deg.T, feat)
    acc = _prop_kernel(src_p, dst_p, fs,
                       jnp.zeros((STG, D2), jnp.float32))
    acc = acc.reshape(NC, N, D2)
    ox, oh = _combine_call(acc[0], acc[1], fs, dsq)
    return (oh, ox)
